# baseline (speedup 1.0000x reference)
import functools

import jax
import jax.numpy as jnp
from jax import lax
from jax.experimental import pallas as pl
from jax.experimental.pallas import tpu as pltpu
from jax.experimental.pallas import tpu_sc as plsc

_N = 10000
_D = 128
_B = 4096
_KV = 11
_KP = 16

_info = plsc.get_sparse_core_info()
_NC = _info.num_cores
_NS = _info.num_subcores
_NW = _NC * _NS
_BW = _B // _NW
_CB = 32
_NCHUNK = _BW // _CB
_CROWS = _CB * _KV
_GL = _BW * _KV
_GLP = _GL + 16
_SROWS = 2 * _CROWS


def _agg_body(samp_hbm, wa_hbm, wf_hbm, feat_hbm, out_hbm,
              samp_v, si_v, wa_v, wf_v, rows_v, out_v,
              sem_r0, sem_r1, sem_w, sem_o):
    wid = lax.axis_index("s") * _NC + lax.axis_index("c")
    base = wid * _BW
    cps = pltpu.make_async_copy(
        samp_hbm.at[pl.ds(base, _BW), :], samp_v, sem_w)
    cpa = pltpu.make_async_copy(
        wa_hbm.at[pl.ds(base * _KP, _BW * _KP)], wa_v, sem_w)
    cpf = pltpu.make_async_copy(
        wf_hbm.at[pl.ds(base * _KP, _BW * _KP)], wf_v, sem_w)
    cps.start()
    cpa.start()
    cpf.start()
    cps.wait()

    si_v[pl.ds(_GLP - _KP, _KP)] = jnp.zeros((_KP,), jnp.int32)

    def build(b, carry):
        si_v[pl.ds(b * _KV, _KP)] = samp_v[b, :]
        return carry

    lax.fori_loop(0, _BW, build, 0)

    def rows_copy(c, slot):
        return pltpu.make_async_copy(
            feat_hbm.at[si_v.at[pl.ds(c * _CROWS, _CROWS)]],
            rows_v.at[pl.ds(slot * _CROWS, _CROWS), :],
            sem_r0 if slot == 0 else sem_r1)

    rows_copy(0, 0).start()
    cpa.wait()
    cpf.wait()

    def out_copy(c):
        return pltpu.make_async_copy(
            out_v.at[pl.ds(c * _CB, _CB), :],
            out_hbm.at[pl.ds(base + c * _CB, _CB), :], sem_o)

    def chunk(c, carry):
        par = lax.rem(c, 2)

        @pl.when(c + 1 < _NCHUNK)
        def _():
            @pl.when(par == 0)
            def _():
                rows_copy(c + 1, 1).start()

            @pl.when(par == 1)
            def _():
                rows_copy(c + 1, 0).start()

        @pl.when(par == 0)
        def _():
            rows_copy(c, 0).wait()

        @pl.when(par == 1)
        def _():
            rows_copy(c, 1).wait()

        soff = par * _CROWS

        def one_row(j, carry2):
            b = c * _CB + j
            w = wa_v[pl.ds(b * _KP, _KP)] + wf_v[pl.ds(b * _KP, _KP)]
            ws = [w[k] for k in range(_KV)]
            s = ws[0]
            for k in range(1, _KV):
                s = s + ws[k]
            inv_v = 1.0 / lax.broadcast(s, (_KP,))
            r = soff + j * _KV
            for d in range(_D // 16):
                acc = ws[0] * rows_v[r, pl.ds(d * 16, 16)]
                for k in range(1, _KV):
                    acc = acc + ws[k] * rows_v[r + k, pl.ds(d * 16, 16)]
                out_v[b, pl.ds(d * 16, 16)] = acc * inv_v
            return carry2

        lax.fori_loop(0, _CB, one_row, 0)
        out_copy(c).start()
        return carry

    lax.fori_loop(0, _NCHUNK, chunk, 0)
    for c in range(_NCHUNK):
        out_copy(c).wait()


_agg = functools.partial(
    pl.kernel,
    out_type=jax.ShapeDtypeStruct((_B, _D), jnp.float32),
    scratch_types=[
        pltpu.VMEM((_BW, _KP), jnp.int32),
        pltpu.VMEM((_GLP,), jnp.int32),
        pltpu.VMEM((_BW * _KP,), jnp.float32),
        pltpu.VMEM((_BW * _KP,), jnp.float32),
        pltpu.VMEM((_SROWS, _D), jnp.float32),
        pltpu.VMEM((_BW, _D), jnp.float32),
        pltpu.SemaphoreType.DMA,
        pltpu.SemaphoreType.DMA,
        pltpu.SemaphoreType.DMA,
        pltpu.SemaphoreType.DMA,
    ],
    mesh=plsc.VectorSubcoreMesh(core_axis_name="c", subcore_axis_name="s"),
)(_agg_body)


def kernel(nodes, neighbors, adj, feat_sims, feat):
    nodes = nodes.astype(jnp.int32)
    neighbors = neighbors.astype(jnp.int32)
    samp = jnp.concatenate(
        [neighbors, nodes[:, None],
         jnp.zeros((_B, _KP - _KV), jnp.int32)], axis=1)
    rows = nodes[:, None]
    wa = adj[rows, samp].reshape(-1)
    wf = feat_sims[rows, samp].reshape(-1)
    return _agg(samp, wa, wf, feat)


# speedup vs baseline: 1.0061x; 1.0061x over previous
import functools

import jax
import jax.numpy as jnp
from jax import lax
from jax.experimental import pallas as pl
from jax.experimental.pallas import tpu as pltpu
from jax.experimental.pallas import tpu_sc as plsc

_N = 10000
_D = 128
_B = 4096
_KV = 11
_KP = 16

_info = plsc.get_sparse_core_info()
_NC = _info.num_cores
_NS = _info.num_subcores
_NW = _NC * _NS
_BW = _B // _NW
_CB = 32
_NCHUNK = _BW // _CB
_CROWS = _CB * _KV
_GL = _BW * _KV
_GLP = _GL + 8
_SROWS = 2 * _CROWS


def _agg_body(si_hbm, wa_hbm, wf_hbm, feat_hbm, out_hbm,
              si_v, wa_v, wf_v, rows_v, out_v,
              sem_r0, sem_r1, sem_w, sem_o):
    wid = lax.axis_index("s") * _NC + lax.axis_index("c")
    base = wid * _BW
    gbase = wid * _GL
    cps = pltpu.make_async_copy(
        si_hbm.at[pl.ds(gbase, _GL)], si_v.at[pl.ds(0, _GL)], sem_w)
    cpa = pltpu.make_async_copy(
        wa_hbm.at[pl.ds(gbase, _GL)], wa_v.at[pl.ds(0, _GL)], sem_w)
    cpf = pltpu.make_async_copy(
        wf_hbm.at[pl.ds(gbase, _GL)], wf_v.at[pl.ds(0, _GL)], sem_w)
    cps.start()
    cpa.start()
    cpf.start()
    cps.wait()

    def rows_copy(c, slot):
        return pltpu.make_async_copy(
            feat_hbm.at[si_v.at[pl.ds(c * _CROWS, _CROWS)]],
            rows_v.at[pl.ds(slot * _CROWS, _CROWS), :],
            sem_r0 if slot == 0 else sem_r1)

    rows_copy(0, 0).start()
    cpa.wait()
    cpf.wait()

    def out_copy(c):
        return pltpu.make_async_copy(
            out_v.at[pl.ds(c * _CB, _CB), :],
            out_hbm.at[pl.ds(base + c * _CB, _CB), :], sem_o)

    def chunk(c, carry):
        par = lax.rem(c, 2)

        @pl.when(c + 1 < _NCHUNK)
        def _():
            @pl.when(par == 0)
            def _():
                rows_copy(c + 1, 1).start()

            @pl.when(par == 1)
            def _():
                rows_copy(c + 1, 0).start()

        @pl.when(par == 0)
        def _():
            rows_copy(c, 0).wait()

        @pl.when(par == 1)
        def _():
            rows_copy(c, 1).wait()

        soff = par * _CROWS

        def one_row(j, carry2):
            b = c * _CB + j
            g = b * _KV
            w = wa_v[pl.ds(g, _KP)] + wf_v[pl.ds(g, _KP)]
            ws = [w[k] for k in range(_KV)]
            s = ws[0]
            for k in range(1, _KV):
                s = s + ws[k]
            inv_v = 1.0 / lax.broadcast(s, (_KP,))
            r = soff + j * _KV
            for d in range(_D // 16):
                acc = ws[0] * rows_v[r, pl.ds(d * 16, 16)]
                for k in range(1, _KV):
                    acc = acc + ws[k] * rows_v[r + k, pl.ds(d * 16, 16)]
                out_v[b, pl.ds(d * 16, 16)] = acc * inv_v
            return carry2

        lax.fori_loop(0, _CB, one_row, 0)
        out_copy(c).start()
        return carry

    lax.fori_loop(0, _NCHUNK, chunk, 0)
    for c in range(_NCHUNK):
        out_copy(c).wait()


_agg = functools.partial(
    pl.kernel,
    out_type=jax.ShapeDtypeStruct((_B, _D), jnp.float32),
    scratch_types=[
        pltpu.VMEM((_GLP,), jnp.int32),
        pltpu.VMEM((_GLP,), jnp.float32),
        pltpu.VMEM((_GLP,), jnp.float32),
        pltpu.VMEM((_SROWS, _D), jnp.float32),
        pltpu.VMEM((_BW, _D), jnp.float32),
        pltpu.SemaphoreType.DMA,
        pltpu.SemaphoreType.DMA,
        pltpu.SemaphoreType.DMA,
        pltpu.SemaphoreType.DMA,
    ],
    mesh=plsc.VectorSubcoreMesh(core_axis_name="c", subcore_axis_name="s"),
)(_agg_body)


def kernel(nodes, neighbors, adj, feat_sims, feat):
    nodes = nodes.astype(jnp.int32)
    neighbors = neighbors.astype(jnp.int32)
    samp = jnp.concatenate([neighbors, nodes[:, None]], axis=1)
    rows = nodes[:, None]
    wa = adj[rows, samp].reshape(-1)
    wf = feat_sims[rows, samp].reshape(-1)
    return _agg(samp.reshape(-1), wa, wf, feat)


# speedup vs baseline: 1.0074x; 1.0013x over previous
import functools

import jax
import jax.numpy as jnp
from jax import lax
from jax.experimental import pallas as pl
from jax.experimental.pallas import tpu as pltpu
from jax.experimental.pallas import tpu_sc as plsc

_N = 10000
_D = 128
_B = 4096
_KV = 11
_KP = 16

_info = plsc.get_sparse_core_info()
_NC = _info.num_cores
_NS = _info.num_subcores
_NW = _NC * _NS
_BW = _B // _NW
_CB = 32
_NCHUNK = _BW // _CB
_CROWS = _CB * _KV
_GL = _BW * _KV
_GLP = _GL + 8
_SROWS = 2 * _CROWS


def _agg_body(si_hbm, wa_hbm, wf_hbm, feat_hbm, out_hbm,
              si_v, wa_v, wf_v, rows_v, out_v,
              sem_r0, sem_r1, sem_w, sem_o):
    wid = lax.axis_index("s") * _NC + lax.axis_index("c")
    base = wid * _BW
    gbase = wid * _GL
    cps = pltpu.make_async_copy(
        si_hbm.at[pl.ds(gbase, _GL)], si_v.at[pl.ds(0, _GL)], sem_w)
    cpa = pltpu.make_async_copy(
        wa_hbm.at[pl.ds(gbase, _GL)], wa_v.at[pl.ds(0, _GL)], sem_w)
    cpf = pltpu.make_async_copy(
        wf_hbm.at[pl.ds(gbase, _GL)], wf_v.at[pl.ds(0, _GL)], sem_w)
    cps.start()
    cpa.start()
    cpf.start()
    cps.wait()

    def rows_copy(c, slot):
        return pltpu.make_async_copy(
            feat_hbm.at[si_v.at[pl.ds(c * _CROWS, _CROWS)]],
            rows_v.at[pl.ds(slot * _CROWS, _CROWS), :],
            sem_r0 if slot == 0 else sem_r1)

    rows_copy(0, 0).start()
    cpa.wait()
    cpf.wait()

    def out_copy(c):
        return pltpu.make_async_copy(
            out_v.at[pl.ds(c * _CB, _CB), :],
            out_hbm.at[pl.ds(base + c * _CB, _CB), :], sem_o)

    def chunk(c, carry):
        par = lax.rem(c, 2)

        @pl.when(c + 1 < _NCHUNK)
        def _():
            @pl.when(par == 0)
            def _():
                rows_copy(c + 1, 1).start()

            @pl.when(par == 1)
            def _():
                rows_copy(c + 1, 0).start()

        @pl.when(par == 0)
        def _():
            rows_copy(c, 0).wait()

        @pl.when(par == 1)
        def _():
            rows_copy(c, 1).wait()

        soff = par * _CROWS

        def one_row(j, carry2):
            b = c * _CB + j
            g = b * _KV
            w = wa_v[pl.ds(g, _KP)] + wf_v[pl.ds(g, _KP)]
            ws = [w[k] for k in range(_KV)]
            s = ws[0]
            for k in range(1, _KV):
                s = s + ws[k]
            inv_v = 1.0 / lax.broadcast(s, (_KP,))
            r = soff + j * _KV
            for d in range(_D // 16):
                acc = ws[0] * rows_v[r, pl.ds(d * 16, 16)]
                for k in range(1, _KV):
                    acc = acc + ws[k] * rows_v[r + k, pl.ds(d * 16, 16)]
                out_v[b, pl.ds(d * 16, 16)] = acc * inv_v
            return carry2

        lax.fori_loop(0, _CB, one_row, 0, unroll=2)
        out_copy(c).start()
        return carry

    lax.fori_loop(0, _NCHUNK, chunk, 0)
    for c in range(_NCHUNK):
        out_copy(c).wait()


_agg = functools.partial(
    pl.kernel,
    out_type=jax.ShapeDtypeStruct((_B, _D), jnp.float32),
    scratch_types=[
        pltpu.VMEM((_GLP,), jnp.int32),
        pltpu.VMEM((_GLP,), jnp.float32),
        pltpu.VMEM((_GLP,), jnp.float32),
        pltpu.VMEM((_SROWS, _D), jnp.float32),
        pltpu.VMEM((_BW, _D), jnp.float32),
        pltpu.SemaphoreType.DMA,
        pltpu.SemaphoreType.DMA,
        pltpu.SemaphoreType.DMA,
        pltpu.SemaphoreType.DMA,
    ],
    mesh=plsc.VectorSubcoreMesh(core_axis_name="c", subcore_axis_name="s"),
)(_agg_body)


def kernel(nodes, neighbors, adj, feat_sims, feat):
    nodes = nodes.astype(jnp.int32)
    neighbors = neighbors.astype(jnp.int32)
    samp = jnp.concatenate([neighbors, nodes[:, None]], axis=1)
    rows = nodes[:, None]
    wa = adj[rows, samp].reshape(-1)
    wf = feat_sims[rows, samp].reshape(-1)
    return _agg(samp.reshape(-1), wa, wf, feat)


# speedup vs baseline: 1.0077x; 1.0003x over previous
import functools

import jax
import jax.numpy as jnp
from jax import lax
from jax.experimental import pallas as pl
from jax.experimental.pallas import tpu as pltpu
from jax.experimental.pallas import tpu_sc as plsc

_N = 10000
_D = 128
_B = 4096
_KV = 11
_KP = 16

_info = plsc.get_sparse_core_info()
_NC = _info.num_cores
_NS = _info.num_subcores
_NW = _NC * _NS
_BW = _B // _NW
_CB = 32
_NCHUNK = _BW // _CB
_CROWS = _CB * _KV
_GL = _BW * _KV
_GLP = _GL + 8
_SROWS = 2 * _CROWS


def _agg_body(si_hbm, wa_hbm, wf_hbm, feat_hbm, out_hbm,
              si_v, wa_v, wf_v, rows_v, out_v,
              sem_r0, sem_r1, sem_w, sem_o):
    wid = lax.axis_index("s") * _NC + lax.axis_index("c")
    base = wid * _BW
    gbase = wid * _GL
    cps = pltpu.make_async_copy(
        si_hbm.at[pl.ds(gbase, _GL)], si_v.at[pl.ds(0, _GL)], sem_w)
    cpa = pltpu.make_async_copy(
        wa_hbm.at[pl.ds(gbase, _GL)], wa_v.at[pl.ds(0, _GL)], sem_w)
    cpf = pltpu.make_async_copy(
        wf_hbm.at[pl.ds(gbase, _GL)], wf_v.at[pl.ds(0, _GL)], sem_w)
    cps.start()
    cpa.start()
    cpf.start()
    cps.wait()

    def rows_copy(c, slot):
        return pltpu.make_async_copy(
            feat_hbm.at[si_v.at[pl.ds(c * _CROWS, _CROWS)]],
            rows_v.at[pl.ds(slot * _CROWS, _CROWS), :],
            sem_r0 if slot == 0 else sem_r1)

    rows_copy(0, 0).start()
    cpa.wait()
    cpf.wait()

    def out_copy(c):
        return pltpu.make_async_copy(
            out_v.at[pl.ds(c * _CB, _CB), :],
            out_hbm.at[pl.ds(base + c * _CB, _CB), :], sem_o)

    def chunk(c, carry):
        par = lax.rem(c, 2)

        @pl.when(c + 1 < _NCHUNK)
        def _():
            @pl.when(par == 0)
            def _():
                rows_copy(c + 1, 1).start()

            @pl.when(par == 1)
            def _():
                rows_copy(c + 1, 0).start()

        @pl.when(par == 0)
        def _():
            rows_copy(c, 0).wait()

        @pl.when(par == 1)
        def _():
            rows_copy(c, 1).wait()

        soff = par * _CROWS

        def one_row(j, carry2):
            b = c * _CB + j
            g = b * _KV
            w = wa_v[pl.ds(g, _KP)] + wf_v[pl.ds(g, _KP)]
            ws = [w[k] for k in range(_KV)]
            s = ws[0]
            for k in range(1, _KV):
                s = s + ws[k]
            inv_v = 1.0 / lax.broadcast(s, (_KP,))
            r = soff + j * _KV
            for d in range(_D // 16):
                acc = ws[0] * rows_v[r, pl.ds(d * 16, 16)]
                for k in range(1, _KV):
                    acc = acc + ws[k] * rows_v[r + k, pl.ds(d * 16, 16)]
                out_v[b, pl.ds(d * 16, 16)] = acc * inv_v
            return carry2

        lax.fori_loop(0, _CB, one_row, 0, unroll=2)
        out_copy(c).start()
        return carry

    lax.fori_loop(0, _NCHUNK, chunk, 0)
    for c in range(_NCHUNK):
        out_copy(c).wait()


_agg = functools.partial(
    pl.kernel,
    out_type=jax.ShapeDtypeStruct((_B, _D), jnp.float32),
    scratch_types=[
        pltpu.VMEM((_GLP,), jnp.int32),
        pltpu.VMEM((_GLP,), jnp.float32),
        pltpu.VMEM((_GLP,), jnp.float32),
        pltpu.VMEM((_SROWS, _D), jnp.float32),
        pltpu.VMEM((_BW, _D), jnp.float32),
        pltpu.SemaphoreType.DMA,
        pltpu.SemaphoreType.DMA,
        pltpu.SemaphoreType.DMA,
        pltpu.SemaphoreType.DMA,
    ],
    mesh=plsc.VectorSubcoreMesh(core_axis_name="c", subcore_axis_name="s"),
)(_agg_body)


def kernel(nodes, neighbors, adj, feat_sims, feat):
    nodes = nodes.astype(jnp.int32)
    neighbors = neighbors.astype(jnp.int32)
    samp = jnp.concatenate([neighbors, nodes[:, None]], axis=1)
    rows = nodes[:, None]
    wa = adj.at[rows, samp].get(mode="promise_in_bounds").reshape(-1)
    wf = feat_sims.at[rows, samp].get(mode="promise_in_bounds").reshape(-1)
    return _agg(samp.reshape(-1), wa, wf, feat)


# speedup vs baseline: 1.0223x; 1.0145x over previous
import functools

import jax
import jax.numpy as jnp
from jax import lax
from jax.experimental import pallas as pl
from jax.experimental.pallas import tpu as pltpu
from jax.experimental.pallas import tpu_sc as plsc

_N = 10000
_D = 128
_B = 4096
_KV = 11
_KP = 16

_info = plsc.get_sparse_core_info()
_NC = _info.num_cores
_NS = _info.num_subcores
_NW = _NC * _NS
_BW = _B // _NW
_CB = 32
_NCHUNK = _BW // _CB
_CROWS = _CB * _KV
_GL = _BW * _KV
_GLP = _GL + 8
_SROWS = 2 * _CROWS


def _agg_body(si_hbm, wa_hbm, wf_hbm, feat_hbm, out_hbm,
              si_v, wa_v, wf_v, rows_v, out_v,
              sem_r0, sem_r1, sem_w, sem_o):
    wid = lax.axis_index("s") * _NC + lax.axis_index("c")
    base = wid * _BW
    gbase = wid * _GL
    cps = pltpu.make_async_copy(
        si_hbm.at[pl.ds(gbase, _GL)], si_v.at[pl.ds(0, _GL)], sem_w)
    cpa = pltpu.make_async_copy(
        wa_hbm.at[pl.ds(gbase, _GL)], wa_v.at[pl.ds(0, _GL)], sem_w)
    cpf = pltpu.make_async_copy(
        wf_hbm.at[pl.ds(gbase, _GL)], wf_v.at[pl.ds(0, _GL)], sem_w)
    cps.start()
    cpa.start()
    cpf.start()
    cps.wait()

    def rows_copy(c, slot):
        return pltpu.make_async_copy(
            feat_hbm.at[si_v.at[pl.ds(c * _CROWS, _CROWS)]],
            rows_v.at[pl.ds(slot * _CROWS, _CROWS), :],
            sem_r0 if slot == 0 else sem_r1)

    rows_copy(0, 0).start()
    cpa.wait()
    cpf.wait()

    def out_copy(c):
        return pltpu.make_async_copy(
            out_v.at[pl.ds(c * _CB, _CB), :],
            out_hbm.at[pl.ds(base + c * _CB, _CB), :], sem_o)

    def chunk(c, carry):
        par = lax.rem(c, 2)

        @pl.when(c + 1 < _NCHUNK)
        def _():
            @pl.when(par == 0)
            def _():
                rows_copy(c + 1, 1).start()

            @pl.when(par == 1)
            def _():
                rows_copy(c + 1, 0).start()

        @pl.when(par == 0)
        def _():
            rows_copy(c, 0).wait()

        @pl.when(par == 1)
        def _():
            rows_copy(c, 1).wait()

        soff = par * _CROWS

        def one_row(j, carry2):
            b = c * _CB + j
            g = b * _KV
            w = wa_v[pl.ds(g, _KP)] + wf_v[pl.ds(g, _KP)]
            ws = [w[k] for k in range(_KV)]
            s = ws[0]
            for k in range(1, _KV):
                s = s + ws[k]
            inv_v = 1.0 / lax.broadcast(s, (_KP,))
            r = soff + j * _KV
            for d in range(_D // 16):
                acc = ws[0] * rows_v[r, pl.ds(d * 16, 16)]
                for k in range(1, _KV):
                    acc = acc + ws[k] * rows_v[r + k, pl.ds(d * 16, 16)]
                out_v[b, pl.ds(d * 16, 16)] = acc * inv_v
            return carry2

        lax.fori_loop(0, _CB, one_row, 0, unroll=2)
        out_copy(c).start()
        return carry

    lax.fori_loop(0, _NCHUNK, chunk, 0)
    for c in range(_NCHUNK):
        out_copy(c).wait()


_agg = functools.partial(
    pl.kernel,
    out_type=jax.ShapeDtypeStruct((_B, _D), jnp.float32),
    scratch_types=[
        pltpu.VMEM((_GLP,), jnp.int32),
        pltpu.VMEM((_GLP,), jnp.float32),
        pltpu.VMEM((_GLP,), jnp.float32),
        pltpu.VMEM((_SROWS, _D), jnp.float32),
        pltpu.VMEM((_BW, _D), jnp.float32),
        pltpu.SemaphoreType.DMA,
        pltpu.SemaphoreType.DMA,
        pltpu.SemaphoreType.DMA,
        pltpu.SemaphoreType.DMA,
    ],
    mesh=plsc.VectorSubcoreMesh(core_axis_name="c", subcore_axis_name="s"),
)(_agg_body)


def kernel(nodes, neighbors, adj, feat_sims, feat):
    nodes = nodes.astype(jnp.int32)
    neighbors = neighbors.astype(jnp.int32)
    samp = jnp.concatenate([neighbors, nodes[:, None]], axis=1)
    si = samp.reshape(-1)
    ri = jnp.repeat(nodes, _KV)
    idx2 = jnp.stack([ri, si], axis=-1)
    dnums = lax.GatherDimensionNumbers(
        offset_dims=(), collapsed_slice_dims=(0, 1), start_index_map=(0, 1))
    wa = lax.gather(adj, idx2, dnums, (1, 1),
                    mode=lax.GatherScatterMode.PROMISE_IN_BOUNDS)
    wf = lax.gather(feat_sims, idx2, dnums, (1, 1),
                    mode=lax.GatherScatterMode.PROMISE_IN_BOUNDS)
    return _agg(si, wa, wf, feat)
